# baseline (speedup 1.0000x reference)
import functools

import jax
import jax.numpy as jnp
from jax.experimental import pallas as pl
from jax.experimental.pallas import tpu as pltpu

_EPS = 1e-5
_BB = 128


def _lrelu(h):
    return jnp.where(h >= 0, h, 0.2 * h)


def _bdot(a, s):
    return jax.lax.dot_general(
        a, s, (((2,), (1,)), ((0,), (0,))), preferred_element_type=jnp.float32
    )


def _first_kernel(x_ref, c_ref, adj_ref, wx_ref, wc_ref, b_ref,
                  z_ref, adjh_ref, ps_ref, pq_ref):
    bb, n, fx = x_ref.shape
    fc = c_ref.shape[-1]
    fo = wx_ref.shape[-1]
    s = jnp.dot(x_ref[...].reshape(bb * n, fx), wx_ref[...],
                preferred_element_type=jnp.float32)
    s = s + jnp.dot(c_ref[...].reshape(bb * n, fc), wc_ref[...],
                    preferred_element_type=jnp.float32)
    a = adj_ref[...]
    adjh_ref[...] = a.astype(jnp.bfloat16)
    z = _bdot(a, s.reshape(bb, n, fo)) + b_ref[...]
    ps_ref[...] = jnp.sum(z, axis=(0, 2)).reshape(1, 1, n)
    pq_ref[...] = jnp.sum(z * z, axis=(0, 2)).reshape(1, 1, n)
    q = bb // 4
    z_ref[...] = jnp.concatenate(
        [z[0 * q:1 * q], z[1 * q:2 * q], z[2 * q:3 * q], z[3 * q:4 * q]],
        axis=2).astype(z_ref.dtype)


def _mid2_kernel(zp_ref, adj_ref, sc_ref, sh_ref, wbd_ref, b_ref,
                 z_ref, ps_ref, pq_ref):
    bq, n, _ = zp_ref.shape
    fo = b_ref.shape[-1]
    h = _lrelu(zp_ref[...].astype(jnp.float32) * sc_ref[...][None]
               + sh_ref[...][None])
    s = jnp.dot(h.reshape(bq * n, 4 * 32), wbd_ref[...],
                preferred_element_type=jnp.float32)
    s = s.reshape(bq, n, 4 * fo)
    a = adj_ref[...].astype(jnp.float32)
    zs = jnp.zeros((1, n), jnp.float32)
    zq = jnp.zeros((1, n), jnp.float32)
    zouts = []
    for g in range(4):
        zg = _bdot(a[g * bq:(g + 1) * bq],
                   s[:, :, g * fo:(g + 1) * fo]) + b_ref[...]
        zs = zs + jnp.sum(zg, axis=(0, 2)).reshape(1, n)
        zq = zq + jnp.sum(zg * zg, axis=(0, 2)).reshape(1, n)
        zouts.append(zg)
    ps_ref[...] = zs.reshape(1, 1, n)
    pq_ref[...] = zq.reshape(1, 1, n)
    z_ref[...] = jnp.concatenate(
        [jnp.concatenate(zouts[:2], axis=0),
         jnp.concatenate(zouts[2:], axis=0)], axis=2).astype(z_ref.dtype)


def _mid3_kernel(zp_ref, adj_ref, sc_ref, sh_ref, wbd_ref, b_ref,
                 z_ref, ps_ref, pq_ref):
    bh, n, _ = zp_ref.shape
    fo = b_ref.shape[-1]
    h = _lrelu(zp_ref[...].astype(jnp.float32) * sc_ref[...][None]
               + sh_ref[...][None])
    s = jnp.dot(h.reshape(bh * n, 2 * 64), wbd_ref[...],
                preferred_element_type=jnp.float32)
    s = s.reshape(bh, n, 2 * fo)
    a = adj_ref[...].astype(jnp.float32)
    zs = jnp.zeros((1, n), jnp.float32)
    zq = jnp.zeros((1, n), jnp.float32)
    zouts = []
    for g in range(2):
        zg = _bdot(a[g * bh:(g + 1) * bh],
                   s[:, :, g * fo:(g + 1) * fo]) + b_ref[...]
        zs = zs + jnp.sum(zg, axis=(0, 2)).reshape(1, n)
        zq = zq + jnp.sum(zg * zg, axis=(0, 2)).reshape(1, n)
        zouts.append(zg)
    ps_ref[...] = zs.reshape(1, 1, n)
    pq_ref[...] = zq.reshape(1, 1, n)
    z_ref[...] = jnp.concatenate(zouts, axis=0).astype(z_ref.dtype)


def _last_gcn_kernel(zp_ref, adj_ref, sc_ref, sh_ref, w_ref, b_ref,
                     z_ref, ps_ref, pq_ref):
    bb, n, fi = zp_ref.shape
    h = _lrelu(zp_ref[...].astype(jnp.float32) * sc_ref[...][None]
               + sh_ref[...][None])
    s = jnp.sum(h * w_ref[...].reshape(1, 1, fi), axis=2)
    z = jnp.sum(adj_ref[...].astype(jnp.float32) * s[:, None, :],
                axis=2) + b_ref[...]
    z_ref[...] = z
    ps_ref[...] = jnp.sum(z, axis=0).reshape(1, 1, n)
    pq_ref[...] = jnp.sum(z * z, axis=0).reshape(1, 1, n)


def _head_kernel(z_ref, sc_ref, sh_ref, w5_ref, b5_ref, o_ref):
    h = _lrelu(z_ref[...] * sc_ref[...] + sh_ref[...])
    o = jnp.sum(h * w5_ref[...], axis=1, keepdims=True) + b5_ref[...]
    o_ref[...] = jax.nn.sigmoid(o)


def _finalize(ps, pq, cnt, g, be):
    s = jnp.sum(ps, axis=0).reshape(-1)
    q = jnp.sum(pq, axis=0).reshape(-1)
    mean = s / cnt
    var = q / cnt - mean * mean
    inv = jax.lax.rsqrt(var + _EPS)
    scale = g * inv
    shift = be - mean * scale
    return scale, shift


def kernel(x, adj, c, W1, b1, W2, b2, W3, b3, W4, b4,
           g1, be1, g2, be2, g3, be3, g4, be4, W5, b5):
    B, N, FX = x.shape
    FC = c.shape[-1]
    nblk = B // _BB
    grid = (nblk,)
    params = pltpu.CompilerParams(dimension_semantics=("parallel",))

    def blk(*shape):
        nd = len(shape)
        return pl.BlockSpec(shape, lambda i: (i,) + (0,) * (nd - 1))

    def full(*shape):
        nd = len(shape)
        return pl.BlockSpec(shape, lambda i: (0,) * nd)

    stats_shape = jax.ShapeDtypeStruct((nblk, 1, N), jnp.float32)
    stats_spec = pl.BlockSpec((1, 1, N), lambda i: (i, 0, 0))

    f1, f2, f3 = W1.shape[1], W2.shape[1], W3.shape[1]

    w2bd = jax.scipy.linalg.block_diag(W2, W2, W2, W2)
    w3bd = jax.scipy.linalg.block_diag(W3, W3)

    z1, adjh, ps, pq = pl.pallas_call(
        _first_kernel,
        grid=grid,
        in_specs=[blk(_BB, N, FX), blk(_BB, N, FC), blk(_BB, N, N),
                  full(FX, f1), full(FC, f1), full(1, f1)],
        out_specs=[blk(_BB // 4, N, 4 * f1), blk(_BB, N, N),
                   stats_spec, stats_spec],
        out_shape=[jax.ShapeDtypeStruct((B // 4, N, 4 * f1), jnp.bfloat16),
                   jax.ShapeDtypeStruct((B, N, N), jnp.bfloat16),
                   stats_shape, stats_shape],
        compiler_params=params,
    )(x, c, adj, W1[:FX], W1[FX:], b1.reshape(1, f1))
    sc1, sh1 = _finalize(ps, pq, B * f1, g1, be1)

    z2, ps, pq = pl.pallas_call(
        _mid2_kernel,
        grid=grid,
        in_specs=[blk(_BB // 4, N, 4 * f1), blk(_BB, N, N),
                  full(N, 1), full(N, 1), full(4 * f1, 4 * f2), full(1, f2)],
        out_specs=[blk(_BB // 2, N, 2 * f2), stats_spec, stats_spec],
        out_shape=[jax.ShapeDtypeStruct((B // 2, N, 2 * f2), jnp.bfloat16),
                   stats_shape, stats_shape],
        compiler_params=params,
    )(z1, adjh, sc1.reshape(N, 1), sh1.reshape(N, 1), w2bd, b2.reshape(1, f2))
    sc2, sh2 = _finalize(ps, pq, B * f2, g2, be2)

    z3, ps, pq = pl.pallas_call(
        _mid3_kernel,
        grid=grid,
        in_specs=[blk(_BB // 2, N, 2 * f2), blk(_BB, N, N),
                  full(N, 1), full(N, 1), full(2 * f2, 2 * f3), full(1, f3)],
        out_specs=[blk(_BB, N, f3), stats_spec, stats_spec],
        out_shape=[jax.ShapeDtypeStruct((B, N, f3), jnp.bfloat16),
                   stats_shape, stats_shape],
        compiler_params=params,
    )(z2, adjh, sc2.reshape(N, 1), sh2.reshape(N, 1), w3bd, b3.reshape(1, f3))
    sc3, sh3 = _finalize(ps, pq, B * f3, g3, be3)

    z4, ps, pq = pl.pallas_call(
        _last_gcn_kernel,
        grid=grid,
        in_specs=[blk(_BB, N, f3), blk(_BB, N, N),
                  full(N, 1), full(N, 1), full(f3, 1), full(1, 1)],
        out_specs=[blk(_BB, N), stats_spec, stats_spec],
        out_shape=[jax.ShapeDtypeStruct((B, N), jnp.float32),
                   stats_shape, stats_shape],
        compiler_params=params,
    )(z3, adjh, sc3.reshape(N, 1), sh3.reshape(N, 1), W4, b4.reshape(1, 1))
    sc4, sh4 = _finalize(ps, pq, B, g4, be4)

    out = pl.pallas_call(
        _head_kernel,
        out_shape=jax.ShapeDtypeStruct((B, 1), jnp.float32),
    )(z4, sc4.reshape(1, N), sh4.reshape(1, N), W5.reshape(1, N),
      b5.reshape(1, 1))
    return out


# speedup vs baseline: 1.0843x; 1.0843x over previous
import functools

import jax
import jax.numpy as jnp
from jax.experimental import pallas as pl
from jax.experimental.pallas import tpu as pltpu

_EPS = 1e-5
_BB = 128


def _lrelu(h):
    return jnp.where(h >= 0, h, 0.2 * h)


def _bdot(a, s):
    return jax.lax.dot_general(
        a, s, (((2,), (1,)), ((0,), (0,))), preferred_element_type=jnp.float32
    )


def _first_kernel(x_ref, c_ref, adj_ref, wx_ref, wc_ref, b_ref,
                  z_ref, ps_ref, pq_ref):
    bb, n, fx = x_ref.shape
    fc = c_ref.shape[-1]
    fo = wx_ref.shape[-1]
    s = jnp.dot(x_ref[...].reshape(bb * n, fx), wx_ref[...],
                preferred_element_type=jnp.float32)
    s = s + jnp.dot(c_ref[...].reshape(bb * n, fc), wc_ref[...],
                    preferred_element_type=jnp.float32)
    z = _bdot(adj_ref[...],
              s.reshape(bb, n, fo).astype(jnp.bfloat16)) + b_ref[...]
    ps_ref[...] = jnp.sum(z, axis=(0, 2)).reshape(1, 1, n)
    pq_ref[...] = jnp.sum(z * z, axis=(0, 2)).reshape(1, 1, n)
    q = bb // 4
    z_ref[...] = jnp.concatenate(
        [z[0 * q:1 * q], z[1 * q:2 * q], z[2 * q:3 * q], z[3 * q:4 * q]],
        axis=2).astype(z_ref.dtype)


def _mid2_kernel(zp_ref, adj_ref, sc_ref, sh_ref, wbd_ref, b_ref,
                 z_ref, ps_ref, pq_ref):
    bq, n, _ = zp_ref.shape
    fo = b_ref.shape[-1]
    h = _lrelu(zp_ref[...].astype(jnp.float32) * sc_ref[...][None]
               + sh_ref[...][None])
    s = jnp.dot(h.reshape(bq * n, 4 * 32), wbd_ref[...],
                preferred_element_type=jnp.float32)
    s = s.reshape(bq, n, 4 * fo).astype(jnp.bfloat16)
    a = adj_ref[...]
    zs = jnp.zeros((1, n), jnp.float32)
    zq = jnp.zeros((1, n), jnp.float32)
    zouts = []
    for g in range(4):
        zg = _bdot(a[g * bq:(g + 1) * bq],
                   s[:, :, g * fo:(g + 1) * fo]) + b_ref[...]
        zs = zs + jnp.sum(zg, axis=(0, 2)).reshape(1, n)
        zq = zq + jnp.sum(zg * zg, axis=(0, 2)).reshape(1, n)
        zouts.append(zg)
    ps_ref[...] = zs.reshape(1, 1, n)
    pq_ref[...] = zq.reshape(1, 1, n)
    z_ref[...] = jnp.concatenate(
        [jnp.concatenate(zouts[:2], axis=0),
         jnp.concatenate(zouts[2:], axis=0)], axis=2).astype(z_ref.dtype)


def _mid3_kernel(zp_ref, adj_ref, sc_ref, sh_ref, wbd_ref, b_ref,
                 z_ref, ps_ref, pq_ref):
    bh, n, _ = zp_ref.shape
    fo = b_ref.shape[-1]
    h = _lrelu(zp_ref[...].astype(jnp.float32) * sc_ref[...][None]
               + sh_ref[...][None])
    s = jnp.dot(h.reshape(bh * n, 2 * 64), wbd_ref[...],
                preferred_element_type=jnp.float32)
    s = s.reshape(bh, n, 2 * fo).astype(jnp.bfloat16)
    a = adj_ref[...]
    zs = jnp.zeros((1, n), jnp.float32)
    zq = jnp.zeros((1, n), jnp.float32)
    zouts = []
    for g in range(2):
        zg = _bdot(a[g * bh:(g + 1) * bh],
                   s[:, :, g * fo:(g + 1) * fo]) + b_ref[...]
        zs = zs + jnp.sum(zg, axis=(0, 2)).reshape(1, n)
        zq = zq + jnp.sum(zg * zg, axis=(0, 2)).reshape(1, n)
        zouts.append(zg)
    ps_ref[...] = zs.reshape(1, 1, n)
    pq_ref[...] = zq.reshape(1, 1, n)
    z_ref[...] = jnp.concatenate(zouts, axis=0).astype(z_ref.dtype)


def _last_gcn_kernel(zp_ref, adj_ref, sc_ref, sh_ref, w_ref, b_ref,
                     z_ref, ps_ref, pq_ref):
    bb, n, fi = zp_ref.shape
    h = _lrelu(zp_ref[...].astype(jnp.float32) * sc_ref[...][None]
               + sh_ref[...][None])
    s = jnp.sum(h * w_ref[...].reshape(1, 1, fi), axis=2)
    z = jnp.sum(adj_ref[...].astype(jnp.float32) * s[:, None, :],
                axis=2) + b_ref[...]
    z_ref[...] = z
    ps_ref[...] = jnp.sum(z, axis=0).reshape(1, 1, n)
    pq_ref[...] = jnp.sum(z * z, axis=0).reshape(1, 1, n)


def _head_kernel(z_ref, sc_ref, sh_ref, w5_ref, b5_ref, o_ref):
    h = _lrelu(z_ref[...] * sc_ref[...] + sh_ref[...])
    o = jnp.sum(h * w5_ref[...], axis=1, keepdims=True) + b5_ref[...]
    o_ref[...] = jax.nn.sigmoid(o)


def _finalize(ps, pq, cnt, g, be):
    s = jnp.sum(ps, axis=0).reshape(-1)
    q = jnp.sum(pq, axis=0).reshape(-1)
    mean = s / cnt
    var = q / cnt - mean * mean
    inv = jax.lax.rsqrt(var + _EPS)
    scale = g * inv
    shift = be - mean * scale
    return scale, shift


def kernel(x, adj, c, W1, b1, W2, b2, W3, b3, W4, b4,
           g1, be1, g2, be2, g3, be3, g4, be4, W5, b5):
    B, N, FX = x.shape
    FC = c.shape[-1]
    nblk = B // _BB
    grid = (nblk,)
    params = pltpu.CompilerParams(dimension_semantics=("parallel",))

    def blk(*shape):
        nd = len(shape)
        return pl.BlockSpec(shape, lambda i: (i,) + (0,) * (nd - 1))

    def full(*shape):
        nd = len(shape)
        return pl.BlockSpec(shape, lambda i: (0,) * nd)

    stats_shape = jax.ShapeDtypeStruct((nblk, 1, N), jnp.float32)
    stats_spec = pl.BlockSpec((1, 1, N), lambda i: (i, 0, 0))

    f1, f2, f3 = W1.shape[1], W2.shape[1], W3.shape[1]

    adjh = adj.astype(jnp.bfloat16)

    w2bd = jax.scipy.linalg.block_diag(W2, W2, W2, W2)
    w3bd = jax.scipy.linalg.block_diag(W3, W3)

    z1, ps, pq = pl.pallas_call(
        _first_kernel,
        grid=grid,
        in_specs=[blk(_BB, N, FX), blk(_BB, N, FC), blk(_BB, N, N),
                  full(FX, f1), full(FC, f1), full(1, f1)],
        out_specs=[blk(_BB // 4, N, 4 * f1), stats_spec, stats_spec],
        out_shape=[jax.ShapeDtypeStruct((B // 4, N, 4 * f1), jnp.bfloat16),
                   stats_shape, stats_shape],
        compiler_params=params,
    )(x, c, adjh, W1[:FX], W1[FX:], b1.reshape(1, f1))
    sc1, sh1 = _finalize(ps, pq, B * f1, g1, be1)

    z2, ps, pq = pl.pallas_call(
        _mid2_kernel,
        grid=grid,
        in_specs=[blk(_BB // 4, N, 4 * f1), blk(_BB, N, N),
                  full(N, 1), full(N, 1), full(4 * f1, 4 * f2), full(1, f2)],
        out_specs=[blk(_BB // 2, N, 2 * f2), stats_spec, stats_spec],
        out_shape=[jax.ShapeDtypeStruct((B // 2, N, 2 * f2), jnp.bfloat16),
                   stats_shape, stats_shape],
        compiler_params=params,
    )(z1, adjh, sc1.reshape(N, 1), sh1.reshape(N, 1), w2bd, b2.reshape(1, f2))
    sc2, sh2 = _finalize(ps, pq, B * f2, g2, be2)

    z3, ps, pq = pl.pallas_call(
        _mid3_kernel,
        grid=grid,
        in_specs=[blk(_BB // 2, N, 2 * f2), blk(_BB, N, N),
                  full(N, 1), full(N, 1), full(2 * f2, 2 * f3), full(1, f3)],
        out_specs=[blk(_BB, N, f3), stats_spec, stats_spec],
        out_shape=[jax.ShapeDtypeStruct((B, N, f3), jnp.bfloat16),
                   stats_shape, stats_shape],
        compiler_params=params,
    )(z2, adjh, sc2.reshape(N, 1), sh2.reshape(N, 1), w3bd, b3.reshape(1, f3))
    sc3, sh3 = _finalize(ps, pq, B * f3, g3, be3)

    z4, ps, pq = pl.pallas_call(
        _last_gcn_kernel,
        grid=grid,
        in_specs=[blk(_BB, N, f3), blk(_BB, N, N),
                  full(N, 1), full(N, 1), full(f3, 1), full(1, 1)],
        out_specs=[blk(_BB, N), stats_spec, stats_spec],
        out_shape=[jax.ShapeDtypeStruct((B, N), jnp.float32),
                   stats_shape, stats_shape],
        compiler_params=params,
    )(z3, adjh, sc3.reshape(N, 1), sh3.reshape(N, 1), W4, b4.reshape(1, 1))
    sc4, sh4 = _finalize(ps, pq, B, g4, be4)

    out = pl.pallas_call(
        _head_kernel,
        out_shape=jax.ShapeDtypeStruct((B, 1), jnp.float32),
    )(z4, sc4.reshape(1, N), sh4.reshape(1, N), W5.reshape(1, N),
      b5.reshape(1, 1))
    return out


# speedup vs baseline: 1.3700x; 1.2635x over previous
import functools

import jax
import jax.numpy as jnp
from jax.experimental import pallas as pl
from jax.experimental.pallas import tpu as pltpu

_EPS = 1e-5
_BB = 128


def _lrelu(h):
    return jnp.where(h >= 0, h, 0.2 * h)


def _bdot(a, s):
    return jax.lax.dot_general(
        a, s, (((2,), (1,)), ((0,), (0,))), preferred_element_type=jnp.float32
    )


def _sup1_kernel(xt_ref, ct_ref, wx_ref, wc_ref, st_ref):
    st = jax.lax.dot_general(
        wx_ref[...], xt_ref[...].astype(jnp.bfloat16),
        (((2,), (1,)), ((0,), (0,))), preferred_element_type=jnp.float32)
    st = st + jax.lax.dot_general(
        wc_ref[...], ct_ref[...].astype(jnp.bfloat16),
        (((2,), (1,)), ((0,), (0,))), preferred_element_type=jnp.float32)
    st_ref[...] = st.astype(st_ref.dtype)


def _first_kernel(sp_ref, adj_ref, b_ref, z_ref, ps_ref, pq_ref):
    bq, n, _ = sp_ref.shape
    fo = b_ref.shape[-1]
    sp = sp_ref[...]
    a = adj_ref[...]
    zs = jnp.zeros((1, n), jnp.float32)
    zq = jnp.zeros((1, n), jnp.float32)
    zouts = []
    for g in range(4):
        zg = _bdot(a[g * bq:(g + 1) * bq],
                   sp[:, :, g * fo:(g + 1) * fo]) + b_ref[...]
        zs = zs + jnp.sum(zg, axis=(0, 2)).reshape(1, n)
        zq = zq + jnp.sum(zg * zg, axis=(0, 2)).reshape(1, n)
        zouts.append(zg)
    ps_ref[...] = zs.reshape(1, 1, n)
    pq_ref[...] = zq.reshape(1, 1, n)
    z_ref[...] = jnp.concatenate(
        [jnp.concatenate([zouts[0], zouts[1]], axis=2),
         jnp.concatenate([zouts[2], zouts[3]], axis=2)],
        axis=2).astype(z_ref.dtype)


def _mid2_kernel(zp_ref, adj_ref, sc_ref, sh_ref, wbd_ref, b_ref,
                 z_ref, ps_ref, pq_ref):
    bq, n, _ = zp_ref.shape
    fo = b_ref.shape[-1]
    h = _lrelu(zp_ref[...].astype(jnp.float32) * sc_ref[...][None]
               + sh_ref[...][None])
    s = jnp.dot(h.reshape(bq * n, 4 * 32), wbd_ref[...],
                preferred_element_type=jnp.float32)
    s = s.reshape(bq, n, 4 * fo).astype(jnp.bfloat16)
    a = adj_ref[...]
    zs = jnp.zeros((1, n), jnp.float32)
    zq = jnp.zeros((1, n), jnp.float32)
    zouts = []
    for g in range(4):
        zg = _bdot(a[g * bq:(g + 1) * bq],
                   s[:, :, g * fo:(g + 1) * fo]) + b_ref[...]
        zs = zs + jnp.sum(zg, axis=(0, 2)).reshape(1, n)
        zq = zq + jnp.sum(zg * zg, axis=(0, 2)).reshape(1, n)
        zouts.append(zg)
    ps_ref[...] = zs.reshape(1, 1, n)
    pq_ref[...] = zq.reshape(1, 1, n)
    z_ref[...] = jnp.concatenate(
        [jnp.concatenate(zouts[:2], axis=0),
         jnp.concatenate(zouts[2:], axis=0)], axis=2).astype(z_ref.dtype)


def _mid3_kernel(zp_ref, adj_ref, sc_ref, sh_ref, wbd_ref, b_ref,
                 z_ref, ps_ref, pq_ref):
    bh, n, _ = zp_ref.shape
    fo = b_ref.shape[-1]
    h = _lrelu(zp_ref[...].astype(jnp.float32) * sc_ref[...][None]
               + sh_ref[...][None])
    s = jnp.dot(h.reshape(bh * n, 2 * 64), wbd_ref[...],
                preferred_element_type=jnp.float32)
    s = s.reshape(bh, n, 2 * fo).astype(jnp.bfloat16)
    a = adj_ref[...]
    zs = jnp.zeros((1, n), jnp.float32)
    zq = jnp.zeros((1, n), jnp.float32)
    zouts = []
    for g in range(2):
        zg = _bdot(a[g * bh:(g + 1) * bh],
                   s[:, :, g * fo:(g + 1) * fo]) + b_ref[...]
        zs = zs + jnp.sum(zg, axis=(0, 2)).reshape(1, n)
        zq = zq + jnp.sum(zg * zg, axis=(0, 2)).reshape(1, n)
        zouts.append(zg)
    ps_ref[...] = zs.reshape(1, 1, n)
    pq_ref[...] = zq.reshape(1, 1, n)
    z_ref[...] = jnp.concatenate(zouts, axis=0).astype(z_ref.dtype)


def _last_gcn_kernel(zp_ref, adj_ref, sc_ref, sh_ref, w_ref, b_ref,
                     z_ref, ps_ref, pq_ref):
    bb, n, fi = zp_ref.shape
    h = _lrelu(zp_ref[...].astype(jnp.float32) * sc_ref[...][None]
               + sh_ref[...][None])
    s = jnp.sum(h * w_ref[...].reshape(1, 1, fi), axis=2)
    z = jnp.sum(adj_ref[...].astype(jnp.float32) * s[:, None, :],
                axis=2) + b_ref[...]
    z_ref[...] = z
    ps_ref[...] = jnp.sum(z, axis=0).reshape(1, 1, n)
    pq_ref[...] = jnp.sum(z * z, axis=0).reshape(1, 1, n)


def _head_kernel(z_ref, sc_ref, sh_ref, w5_ref, b5_ref, o_ref):
    h = _lrelu(z_ref[...] * sc_ref[...] + sh_ref[...])
    o = jnp.sum(h * w5_ref[...], axis=1, keepdims=True) + b5_ref[...]
    o_ref[...] = jax.nn.sigmoid(o)


def _finalize(ps, pq, cnt, g, be):
    s = jnp.sum(ps, axis=0).reshape(-1)
    q = jnp.sum(pq, axis=0).reshape(-1)
    mean = s / cnt
    var = q / cnt - mean * mean
    inv = jax.lax.rsqrt(var + _EPS)
    scale = g * inv
    shift = be - mean * scale
    return scale, shift


def kernel(x, adj, c, W1, b1, W2, b2, W3, b3, W4, b4,
           g1, be1, g2, be2, g3, be3, g4, be4, W5, b5):
    B, N, FX = x.shape
    FC = c.shape[-1]
    nblk = B // _BB
    grid = (nblk,)
    params = pltpu.CompilerParams(dimension_semantics=("parallel",))

    def blk(*shape):
        nd = len(shape)
        return pl.BlockSpec(shape, lambda i: (i,) + (0,) * (nd - 1))

    def full(*shape):
        nd = len(shape)
        return pl.BlockSpec(shape, lambda i: (0,) * nd)

    stats_shape = jax.ShapeDtypeStruct((nblk, 1, N), jnp.float32)
    stats_spec = pl.BlockSpec((1, 1, N), lambda i: (i, 0, 0))

    f1, f2, f3 = W1.shape[1], W2.shape[1], W3.shape[1]

    adjh = adj.astype(jnp.bfloat16)

    w2bd = jax.scipy.linalg.block_diag(W2, W2, W2, W2)
    w3bd = jax.scipy.linalg.block_diag(W3, W3)

    xt = jnp.transpose(x, (1, 2, 0))
    ct = jnp.transpose(c, (1, 2, 0))
    wxb = jnp.broadcast_to(W1[:FX].T[None].astype(jnp.bfloat16), (N, f1, FX))
    wcb = jnp.broadcast_to(W1[FX:].T[None].astype(jnp.bfloat16), (N, f1, FC))
    BL = B // 8
    s1t = pl.pallas_call(
        _sup1_kernel,
        grid=(8,),
        in_specs=[pl.BlockSpec((N, FX, BL), lambda i: (0, 0, i)),
                  pl.BlockSpec((N, FC, BL), lambda i: (0, 0, i)),
                  full(N, f1, FX), full(N, f1, FC)],
        out_specs=pl.BlockSpec((N, f1, BL), lambda i: (0, 0, i)),
        out_shape=jax.ShapeDtypeStruct((N, f1, B), jnp.bfloat16),
        compiler_params=params,
    )(xt, ct, wxb, wcb)
    s1p = (s1t.transpose(2, 0, 1)
           .reshape(nblk, 4, _BB // 4, N, f1)
           .transpose(0, 2, 3, 1, 4)
           .reshape(B // 4, N, 4 * f1))

    z1, ps, pq = pl.pallas_call(
        _first_kernel,
        grid=grid,
        in_specs=[blk(_BB // 4, N, 4 * f1), blk(_BB, N, N), full(1, f1)],
        out_specs=[blk(_BB // 4, N, 4 * f1), stats_spec, stats_spec],
        out_shape=[jax.ShapeDtypeStruct((B // 4, N, 4 * f1), jnp.bfloat16),
                   stats_shape, stats_shape],
        compiler_params=params,
    )(s1p, adjh, b1.reshape(1, f1))
    sc1, sh1 = _finalize(ps, pq, B * f1, g1, be1)

    z2, ps, pq = pl.pallas_call(
        _mid2_kernel,
        grid=grid,
        in_specs=[blk(_BB // 4, N, 4 * f1), blk(_BB, N, N),
                  full(N, 1), full(N, 1), full(4 * f1, 4 * f2), full(1, f2)],
        out_specs=[blk(_BB // 2, N, 2 * f2), stats_spec, stats_spec],
        out_shape=[jax.ShapeDtypeStruct((B // 2, N, 2 * f2), jnp.bfloat16),
                   stats_shape, stats_shape],
        compiler_params=params,
    )(z1, adjh, sc1.reshape(N, 1), sh1.reshape(N, 1), w2bd, b2.reshape(1, f2))
    sc2, sh2 = _finalize(ps, pq, B * f2, g2, be2)

    z3, ps, pq = pl.pallas_call(
        _mid3_kernel,
        grid=grid,
        in_specs=[blk(_BB // 2, N, 2 * f2), blk(_BB, N, N),
                  full(N, 1), full(N, 1), full(2 * f2, 2 * f3), full(1, f3)],
        out_specs=[blk(_BB, N, f3), stats_spec, stats_spec],
        out_shape=[jax.ShapeDtypeStruct((B, N, f3), jnp.bfloat16),
                   stats_shape, stats_shape],
        compiler_params=params,
    )(z2, adjh, sc2.reshape(N, 1), sh2.reshape(N, 1), w3bd, b3.reshape(1, f3))
    sc3, sh3 = _finalize(ps, pq, B * f3, g3, be3)

    z4, ps, pq = pl.pallas_call(
        _last_gcn_kernel,
        grid=grid,
        in_specs=[blk(_BB, N, f3), blk(_BB, N, N),
                  full(N, 1), full(N, 1), full(f3, 1), full(1, 1)],
        out_specs=[blk(_BB, N), stats_spec, stats_spec],
        out_shape=[jax.ShapeDtypeStruct((B, N), jnp.float32),
                   stats_shape, stats_shape],
        compiler_params=params,
    )(z3, adjh, sc3.reshape(N, 1), sh3.reshape(N, 1), W4, b4.reshape(1, 1))
    sc4, sh4 = _finalize(ps, pq, B, g4, be4)

    out = pl.pallas_call(
        _head_kernel,
        out_shape=jax.ShapeDtypeStruct((B, 1), jnp.float32),
    )(z4, sc4.reshape(1, N), sh4.reshape(1, N), W5.reshape(1, N),
      b5.reshape(1, 1))
    return out


# speedup vs baseline: 1.3840x; 1.0102x over previous
import functools

import jax
import jax.numpy as jnp
from jax.experimental import pallas as pl
from jax.experimental.pallas import tpu as pltpu

_EPS = 1e-5
_BB = 128


def _lrelu(h):
    return jnp.where(h >= 0, h, 0.2 * h)


def _bdot(a, s):
    return jax.lax.dot_general(
        a, s, (((2,), (1,)), ((0,), (0,))), preferred_element_type=jnp.float32
    )


def _sup1_kernel(xt_ref, ct_ref, wx_ref, wc_ref, st_ref):
    st = jax.lax.dot_general(
        wx_ref[...], xt_ref[...].astype(jnp.bfloat16),
        (((2,), (1,)), ((0,), (0,))), preferred_element_type=jnp.float32)
    st = st + jax.lax.dot_general(
        wc_ref[...], ct_ref[...].astype(jnp.bfloat16),
        (((2,), (1,)), ((0,), (0,))), preferred_element_type=jnp.float32)
    st_ref[...] = st.astype(st_ref.dtype)


def _first_kernel(sp_ref, adj_ref, b_ref, z_ref, ps_ref, pq_ref):
    bq, n, _ = sp_ref.shape
    fo = b_ref.shape[-1]
    sp = sp_ref[...]
    a = adj_ref[...]
    zs = jnp.zeros((1, n), jnp.float32)
    zq = jnp.zeros((1, n), jnp.float32)
    zouts = []
    for g in range(4):
        zg = _bdot(a[g * bq:(g + 1) * bq],
                   sp[:, :, g * fo:(g + 1) * fo]) + b_ref[...]
        zs = zs + jnp.sum(zg, axis=(0, 2)).reshape(1, n)
        zq = zq + jnp.sum(zg * zg, axis=(0, 2)).reshape(1, n)
        zouts.append(zg)
    ps_ref[...] = zs.reshape(1, 1, n)
    pq_ref[...] = zq.reshape(1, 1, n)
    z_ref[...] = jnp.concatenate(
        [jnp.concatenate([zouts[0], zouts[1]], axis=2),
         jnp.concatenate([zouts[2], zouts[3]], axis=2)],
        axis=2).astype(z_ref.dtype)


def _mid2_kernel(zp_ref, adj_ref, sc_ref, sh_ref, wbd_ref, b_ref,
                 z_ref, ps_ref, pq_ref):
    bq, n, _ = zp_ref.shape
    fo = b_ref.shape[-1]
    h = _lrelu(zp_ref[...].astype(jnp.float32) * sc_ref[...][None]
               + sh_ref[...][None])
    s = jnp.dot(h.reshape(bq * n, 4 * 32), wbd_ref[...],
                preferred_element_type=jnp.float32)
    s = s.reshape(bq, n, 4 * fo).astype(jnp.bfloat16)
    a = adj_ref[...]
    zs = jnp.zeros((1, n), jnp.float32)
    zq = jnp.zeros((1, n), jnp.float32)
    zouts = []
    for g in range(4):
        zg = _bdot(a[g * bq:(g + 1) * bq],
                   s[:, :, g * fo:(g + 1) * fo]) + b_ref[...]
        zs = zs + jnp.sum(zg, axis=(0, 2)).reshape(1, n)
        zq = zq + jnp.sum(zg * zg, axis=(0, 2)).reshape(1, n)
        zouts.append(zg)
    ps_ref[...] = zs.reshape(1, 1, n)
    pq_ref[...] = zq.reshape(1, 1, n)
    z_ref[...] = jnp.concatenate(
        [jnp.concatenate(zouts[:2], axis=0),
         jnp.concatenate(zouts[2:], axis=0)], axis=2).astype(z_ref.dtype)


def _mid3_kernel(zp_ref, adj_ref, sc_ref, sh_ref, wbd_ref, b_ref,
                 z_ref, ps_ref, pq_ref):
    bh, n, _ = zp_ref.shape
    fo = b_ref.shape[-1]
    h = _lrelu(zp_ref[...].astype(jnp.float32) * sc_ref[...][None]
               + sh_ref[...][None])
    s = jnp.dot(h.reshape(bh * n, 2 * 64), wbd_ref[...],
                preferred_element_type=jnp.float32)
    s = s.reshape(bh, n, 2 * fo).astype(jnp.bfloat16)
    a = adj_ref[...]
    zs = jnp.zeros((1, n), jnp.float32)
    zq = jnp.zeros((1, n), jnp.float32)
    zouts = []
    for g in range(2):
        zg = _bdot(a[g * bh:(g + 1) * bh],
                   s[:, :, g * fo:(g + 1) * fo]) + b_ref[...]
        zs = zs + jnp.sum(zg, axis=(0, 2)).reshape(1, n)
        zq = zq + jnp.sum(zg * zg, axis=(0, 2)).reshape(1, n)
        zouts.append(zg)
    ps_ref[...] = zs.reshape(1, 1, n)
    pq_ref[...] = zq.reshape(1, 1, n)
    z_ref[...] = jnp.concatenate(zouts, axis=0).astype(z_ref.dtype)


def _last_gcn_kernel(zp_ref, adj_ref, sc_ref, sh_ref, w_ref, b_ref,
                     z_ref, ps_ref, pq_ref):
    bb, n, fi = zp_ref.shape
    h = _lrelu(zp_ref[...].astype(jnp.float32) * sc_ref[...][None]
               + sh_ref[...][None])
    t = _bdot(adj_ref[...], h.astype(jnp.bfloat16))
    z = jnp.sum(t * w_ref[...].reshape(1, 1, fi), axis=2) + b_ref[...]
    z_ref[...] = z
    ps_ref[...] = jnp.sum(z, axis=0).reshape(1, 1, n)
    pq_ref[...] = jnp.sum(z * z, axis=0).reshape(1, 1, n)


def _head_kernel(z_ref, sc_ref, sh_ref, w5_ref, b5_ref, o_ref):
    h = _lrelu(z_ref[...] * sc_ref[...] + sh_ref[...])
    o = jnp.sum(h * w5_ref[...], axis=1, keepdims=True) + b5_ref[...]
    o_ref[...] = jax.nn.sigmoid(o)


def _finalize(ps, pq, cnt, g, be):
    s = jnp.sum(ps, axis=0).reshape(-1)
    q = jnp.sum(pq, axis=0).reshape(-1)
    mean = s / cnt
    var = q / cnt - mean * mean
    inv = jax.lax.rsqrt(var + _EPS)
    scale = g * inv
    shift = be - mean * scale
    return scale, shift


def kernel(x, adj, c, W1, b1, W2, b2, W3, b3, W4, b4,
           g1, be1, g2, be2, g3, be3, g4, be4, W5, b5):
    B, N, FX = x.shape
    FC = c.shape[-1]
    nblk = B // _BB
    grid = (nblk,)
    params = pltpu.CompilerParams(dimension_semantics=("parallel",))

    def blk(*shape):
        nd = len(shape)
        return pl.BlockSpec(shape, lambda i: (i,) + (0,) * (nd - 1))

    def full(*shape):
        nd = len(shape)
        return pl.BlockSpec(shape, lambda i: (0,) * nd)

    stats_shape = jax.ShapeDtypeStruct((nblk, 1, N), jnp.float32)
    stats_spec = pl.BlockSpec((1, 1, N), lambda i: (i, 0, 0))

    f1, f2, f3 = W1.shape[1], W2.shape[1], W3.shape[1]

    adjh = adj.astype(jnp.bfloat16)

    w2bd = jax.scipy.linalg.block_diag(W2, W2, W2, W2)
    w3bd = jax.scipy.linalg.block_diag(W3, W3)

    xt = jnp.transpose(x, (1, 2, 0))
    ct = jnp.transpose(c, (1, 2, 0))
    wxb = jnp.broadcast_to(W1[:FX].T[None].astype(jnp.bfloat16), (N, f1, FX))
    wcb = jnp.broadcast_to(W1[FX:].T[None].astype(jnp.bfloat16), (N, f1, FC))
    BL = B // 8
    s1t = pl.pallas_call(
        _sup1_kernel,
        grid=(8,),
        in_specs=[pl.BlockSpec((N, FX, BL), lambda i: (0, 0, i)),
                  pl.BlockSpec((N, FC, BL), lambda i: (0, 0, i)),
                  full(N, f1, FX), full(N, f1, FC)],
        out_specs=pl.BlockSpec((N, f1, BL), lambda i: (0, 0, i)),
        out_shape=jax.ShapeDtypeStruct((N, f1, B), jnp.bfloat16),
        compiler_params=params,
    )(xt, ct, wxb, wcb)
    s1p = (s1t.transpose(2, 0, 1)
           .reshape(nblk, 4, _BB // 4, N, f1)
           .transpose(0, 2, 3, 1, 4)
           .reshape(B // 4, N, 4 * f1))

    z1, ps, pq = pl.pallas_call(
        _first_kernel,
        grid=grid,
        in_specs=[blk(_BB // 4, N, 4 * f1), blk(_BB, N, N), full(1, f1)],
        out_specs=[blk(_BB // 4, N, 4 * f1), stats_spec, stats_spec],
        out_shape=[jax.ShapeDtypeStruct((B // 4, N, 4 * f1), jnp.bfloat16),
                   stats_shape, stats_shape],
        compiler_params=params,
    )(s1p, adjh, b1.reshape(1, f1))
    sc1, sh1 = _finalize(ps, pq, B * f1, g1, be1)

    z2, ps, pq = pl.pallas_call(
        _mid2_kernel,
        grid=grid,
        in_specs=[blk(_BB // 4, N, 4 * f1), blk(_BB, N, N),
                  full(N, 1), full(N, 1), full(4 * f1, 4 * f2), full(1, f2)],
        out_specs=[blk(_BB // 2, N, 2 * f2), stats_spec, stats_spec],
        out_shape=[jax.ShapeDtypeStruct((B // 2, N, 2 * f2), jnp.bfloat16),
                   stats_shape, stats_shape],
        compiler_params=params,
    )(z1, adjh, sc1.reshape(N, 1), sh1.reshape(N, 1), w2bd, b2.reshape(1, f2))
    sc2, sh2 = _finalize(ps, pq, B * f2, g2, be2)

    z3, ps, pq = pl.pallas_call(
        _mid3_kernel,
        grid=grid,
        in_specs=[blk(_BB // 2, N, 2 * f2), blk(_BB, N, N),
                  full(N, 1), full(N, 1), full(2 * f2, 2 * f3), full(1, f3)],
        out_specs=[blk(_BB, N, f3), stats_spec, stats_spec],
        out_shape=[jax.ShapeDtypeStruct((B, N, f3), jnp.bfloat16),
                   stats_shape, stats_shape],
        compiler_params=params,
    )(z2, adjh, sc2.reshape(N, 1), sh2.reshape(N, 1), w3bd, b3.reshape(1, f3))
    sc3, sh3 = _finalize(ps, pq, B * f3, g3, be3)

    z4, ps, pq = pl.pallas_call(
        _last_gcn_kernel,
        grid=grid,
        in_specs=[blk(_BB, N, f3), blk(_BB, N, N),
                  full(N, 1), full(N, 1), full(f3, 1), full(1, 1)],
        out_specs=[blk(_BB, N), stats_spec, stats_spec],
        out_shape=[jax.ShapeDtypeStruct((B, N), jnp.float32),
                   stats_shape, stats_shape],
        compiler_params=params,
    )(z3, adjh, sc3.reshape(N, 1), sh3.reshape(N, 1), W4, b4.reshape(1, 1))
    sc4, sh4 = _finalize(ps, pq, B, g4, be4)

    out = pl.pallas_call(
        _head_kernel,
        out_shape=jax.ShapeDtypeStruct((B, 1), jnp.float32),
    )(z4, sc4.reshape(1, N), sh4.reshape(1, N), W5.reshape(1, N),
      b5.reshape(1, 1))
    return out
